# Initial kernel scaffold; baseline (speedup 1.0000x reference)
import jax
import jax.numpy as jnp
from jax.experimental import pallas as pl


def kernel(x):
    raise NotImplementedError("write your pallas kernel here")


# speedup vs baseline: 4.3186x; 4.3186x over previous
import functools

import jax
import jax.numpy as jnp
from jax.experimental import pallas as pl
from jax.experimental.pallas import tpu as pltpu

K_NB = 16
K_FULL = 32
BR = 128
LANES = 128


def _knn_block(x_row_ref, xt_ref, out_ref, dist_ref, *, n_valid, np_cols):
    xr = x_row_ref[...]
    xt = xt_ref[...]

    sq_r = jnp.sum(xr * xr, axis=1, keepdims=True)
    sq_c = jnp.sum(xt * xt, axis=0, keepdims=True)

    s = jnp.dot(xr, xt, preferred_element_type=jnp.float32)

    col = jax.lax.broadcasted_iota(jnp.int32, (BR, np_cols), 1)
    d2 = (sq_r + sq_c) - 2.0 * s
    dist_ref[...] = jnp.where(col < n_valid, d2, jnp.inf)

    iota16 = jax.lax.broadcasted_iota(jnp.int32, (BR, K_NB), 1)

    def body(k, carry):
        c_prev, acc = carry
        d = dist_ref[...]
        d = jnp.where(col == c_prev, jnp.inf, d)
        dist_ref[...] = d
        m = jnp.min(d, axis=1, keepdims=True)
        cand = jnp.where(d == m, col, jnp.int32(2**30))
        c = jnp.min(cand, axis=1, keepdims=True)
        keep = (k % 2 == 0) & (iota16 == (k // 2))
        acc = jnp.where(keep, c, acc)
        return c, acc

    acc0 = jnp.zeros((BR, K_NB), jnp.int32)
    cm1 = jnp.full((BR, 1), -1, jnp.int32)
    _, acc = jax.lax.fori_loop(0, K_FULL, body, (cm1, acc0))
    out_ref[...] = acc


@jax.jit
def kernel(x):
    n = x.shape[0]
    np_cols = ((n + LANES - 1) // LANES) * LANES
    nr = ((n + BR - 1) // BR) * BR
    xp = jnp.zeros((max(nr, np_cols), 3), x.dtype).at[:n].set(x)
    x_rows = xp[:nr]
    xt = xp[:np_cols].T

    grid = nr // BR
    nbr = pl.pallas_call(
        functools.partial(_knn_block, n_valid=n, np_cols=np_cols),
        grid=(grid,),
        in_specs=[
            pl.BlockSpec((BR, 3), lambda i: (i, 0)),
            pl.BlockSpec((3, np_cols), lambda i: (0, 0)),
        ],
        out_specs=pl.BlockSpec((BR, K_NB), lambda i: (i, 0)),
        out_shape=jax.ShapeDtypeStruct((nr, K_NB), jnp.int32),
        scratch_shapes=[pltpu.VMEM((BR, np_cols), jnp.float32)],
        compiler_params=pltpu.CompilerParams(
            dimension_semantics=("parallel",),
        ),
    )(x_rows, xt)

    nbr = nbr[:n]
    center = jnp.broadcast_to(
        jnp.arange(n, dtype=jnp.int32)[:, None], (n, K_NB))
    return jnp.stack([nbr.reshape(-1), center.reshape(-1)], axis=0)


# speedup vs baseline: 7.5439x; 1.7468x over previous
import functools

import jax
import jax.numpy as jnp
from jax.experimental import pallas as pl
from jax.experimental.pallas import tpu as pltpu

K_NB = 16
K_FULL = 32
BR = 128
LANES = 128
CAP = 6
SUB = 8
BIGC = 2**30


def _insert(d, colc, ms, cs):
    bs = [d < m for m in ms]
    nms, ncs = [], []
    for j in range(len(ms)):
        if j == 0:
            nms.append(jnp.where(bs[0], d, ms[0]))
            ncs.append(jnp.where(bs[0], colc, cs[0]))
        else:
            nms.append(jnp.where(bs[j], jnp.where(bs[j - 1], ms[j - 1], d), ms[j]))
            ncs.append(jnp.where(bs[j], jnp.where(bs[j - 1], cs[j - 1], colc), cs[j]))
    return tuple(nms), tuple(ncs)


def _knn_block(x_row_ref, xt_ref, out_ref, dist_ref, mtop_ref, ctop_ref,
               *, n_valid, np_cols):
    nc = np_cols // LANES
    xr = x_row_ref[...]
    xt = xt_ref[...]

    sq_r = jnp.sum(xr * xr, axis=1, keepdims=True)
    sq_c = jnp.sum(xt * xt, axis=0, keepdims=True)
    s = jnp.dot(xr, xt, preferred_element_type=jnp.float32)

    col_full = jax.lax.broadcasted_iota(jnp.int32, (BR, np_cols), 1)
    d2 = (sq_r + sq_c) - 2.0 * s
    d2 = jnp.where(col_full < n_valid, d2, jnp.inf)
    for c in range(nc):
        dist_ref[c] = d2[:, c * LANES:(c + 1) * LANES]

    lane_s = jax.lax.broadcasted_iota(jnp.int32, (SUB, LANES), 1)
    for g in range(BR // SUB):
        r0 = g * SUB

        def stream_body(c, carry, r0=r0):
            ms, cs = carry
            d = dist_ref[c, r0:r0 + SUB, :]
            colc = c * LANES + lane_s
            return _insert(d, colc, ms, cs)

        ms0 = tuple(jnp.full((SUB, LANES), jnp.inf, jnp.float32)
                    for _ in range(CAP))
        cs0 = tuple(jnp.full((SUB, LANES), BIGC, jnp.int32)
                    for _ in range(CAP))
        ms, cs = jax.lax.fori_loop(0, nc, stream_body, (ms0, cs0))
        for j in range(CAP):
            mtop_ref[j, r0:r0 + SUB, :] = ms[j]
            ctop_ref[j, r0:r0 + SUB, :] = cs[j]

    lane_b = jax.lax.broadcasted_iota(jnp.int32, (BR, LANES), 1)
    iota32 = jax.lax.broadcasted_iota(jnp.int32, (BR, K_FULL), 1)

    def refill(ms, cs, acc):
        def body(c, carry):
            ms, cs = carry
            d = dist_ref[c]
            colc = c * LANES + lane_b
            excl = colc == acc[:, 0:1]
            for j in range(1, K_FULL):
                excl = excl | (colc == acc[:, j:j + 1])
            d = jnp.where(excl, jnp.inf, d)
            return _insert(d, colc, ms, cs)

        ms0 = tuple(jnp.full((BR, LANES), jnp.inf, jnp.float32)
                    for _ in range(CAP))
        cs0 = tuple(jnp.full((BR, LANES), BIGC, jnp.int32)
                    for _ in range(CAP))
        return jax.lax.fori_loop(0, nc, body, (ms0, cs0))

    def ext_body(k, carry):
        ms, cs, acc = carry
        exhausted = jnp.any(ms[0] == jnp.inf)
        ms, cs = jax.lax.cond(
            exhausted,
            lambda ms, cs, acc: refill(ms, cs, acc),
            lambda ms, cs, acc: (ms, cs),
            ms, cs, acc)
        m = jnp.min(ms[0], axis=1, keepdims=True)
        colpick = jnp.min(jnp.where(ms[0] == m, cs[0], BIGC),
                          axis=1, keepdims=True)
        acc = jnp.where(iota32 == k, colpick, acc)
        hit = cs[0] == colpick
        nms = tuple(jnp.where(hit, ms[j + 1], ms[j]) for j in range(CAP - 1)) \
            + (jnp.where(hit, jnp.inf, ms[CAP - 1]),)
        ncs = tuple(jnp.where(hit, cs[j + 1], cs[j]) for j in range(CAP - 1)) \
            + (jnp.where(hit, BIGC, cs[CAP - 1]),)
        return nms, ncs, acc

    ms = tuple(mtop_ref[j] for j in range(CAP))
    cs = tuple(ctop_ref[j] for j in range(CAP))
    acc0 = jnp.full((BR, K_FULL), -1, jnp.int32)
    _, _, acc = jax.lax.fori_loop(0, K_FULL, ext_body, (ms, cs, acc0))

    evens = jnp.concatenate([acc[:, 2 * j:2 * j + 1] for j in range(K_NB)],
                            axis=1)
    out_ref[...] = evens


@jax.jit
def kernel(x):
    n = x.shape[0]
    np_cols = ((n + LANES - 1) // LANES) * LANES
    nr = ((n + BR - 1) // BR) * BR
    nc = np_cols // LANES
    xp = jnp.zeros((max(nr, np_cols), 3), x.dtype).at[:n].set(x)
    x_rows = xp[:nr]
    xt = xp[:np_cols].T

    grid = nr // BR
    nbr = pl.pallas_call(
        functools.partial(_knn_block, n_valid=n, np_cols=np_cols),
        grid=(grid,),
        in_specs=[
            pl.BlockSpec((BR, 3), lambda i: (i, 0)),
            pl.BlockSpec((3, np_cols), lambda i: (0, 0)),
        ],
        out_specs=pl.BlockSpec((BR, K_NB), lambda i: (i, 0)),
        out_shape=jax.ShapeDtypeStruct((nr, K_NB), jnp.int32),
        scratch_shapes=[
            pltpu.VMEM((nc, BR, LANES), jnp.float32),
            pltpu.VMEM((CAP, BR, LANES), jnp.float32),
            pltpu.VMEM((CAP, BR, LANES), jnp.int32),
        ],
        compiler_params=pltpu.CompilerParams(
            dimension_semantics=("parallel",),
        ),
    )(x_rows, xt)

    nbr = nbr[:n]
    center = jnp.broadcast_to(
        jnp.arange(n, dtype=jnp.int32)[:, None], (n, K_NB))
    return jnp.stack([nbr.reshape(-1), center.reshape(-1)], axis=0)


# speedup vs baseline: 11.2725x; 1.4943x over previous
import functools

import jax
import jax.numpy as jnp
from jax.experimental import pallas as pl
from jax.experimental.pallas import tpu as pltpu

K_NB = 16
K_FULL = 32
BR = 128
LANES = 128
CAP = 6
SUB = 8
UNROLL = 4
BIGC = 2**30


def _insert(d, colc, ms, cs):
    bs = [d < m for m in ms]
    nms, ncs = [], []
    for j in range(len(ms)):
        if j == 0:
            nms.append(jnp.where(bs[0], d, ms[0]))
            ncs.append(jnp.where(bs[0], colc, cs[0]))
        else:
            nms.append(jnp.where(bs[j], jnp.where(bs[j - 1], ms[j - 1], d), ms[j]))
            ncs.append(jnp.where(bs[j], jnp.where(bs[j - 1], cs[j - 1], colc), cs[j]))
    return tuple(nms), tuple(ncs)


def _build_cache(dist_ref, r0, lane_s, nc, excl_fn=None):
    def stream_body(i, carry):
        ms, cs = carry
        for u in range(UNROLL):
            c = i * UNROLL + u
            d = dist_ref[c, r0:r0 + SUB, :]
            colc = c * LANES + lane_s
            if excl_fn is not None:
                d = jnp.where(excl_fn(colc), jnp.inf, d)
            ms, cs = _insert(d, colc, ms, cs)
        return ms, cs

    ms0 = tuple(jnp.full((SUB, LANES), jnp.inf, jnp.float32)
                for _ in range(CAP))
    cs0 = tuple(jnp.full((SUB, LANES), BIGC, jnp.int32) for _ in range(CAP))
    return jax.lax.fori_loop(0, nc // UNROLL, stream_body, (ms0, cs0))


def _knn_block(x_row_ref, xt_ref, out_ref, dist_ref, mtop_ref, ctop_ref,
               mh_ref, ch_ref, ptr_ref, acc_ref, *, n_valid, np_cols):
    nc = np_cols // LANES
    xr = x_row_ref[...]
    xt = xt_ref[...]

    sq_r = jnp.sum(xr * xr, axis=1, keepdims=True)
    sq_c = jnp.sum(xt * xt, axis=0, keepdims=True)
    s = jnp.dot(xr, xt, preferred_element_type=jnp.float32)

    col_full = jax.lax.broadcasted_iota(jnp.int32, (BR, np_cols), 1)
    d2 = (sq_r + sq_c) - 2.0 * s
    d2 = jnp.where(col_full < n_valid, d2, jnp.inf)
    for c in range(nc):
        dist_ref[c] = d2[:, c * LANES:(c + 1) * LANES]

    lane_s = jax.lax.broadcasted_iota(jnp.int32, (SUB, LANES), 1)
    for g in range(BR // SUB):
        r0 = g * SUB
        ms, cs = _build_cache(dist_ref, r0, lane_s, nc)
        for j in range(CAP):
            mtop_ref[j, r0:r0 + SUB, :] = ms[j]
            ctop_ref[j, r0:r0 + SUB, :] = cs[j]

    mh_ref[...] = mtop_ref[0]
    ch_ref[...] = ctop_ref[0]
    ptr_ref[...] = jnp.zeros((BR, LANES), jnp.int32)
    acc_ref[...] = jnp.full((BR, K_FULL), -1, jnp.int32)

    iota32 = jax.lax.broadcasted_iota(jnp.int32, (BR, K_FULL), 1)

    def ext_body(k, carry):
        exhausted = jnp.any(mh_ref[...] == jnp.inf)

        @pl.when(exhausted)
        def _():
            for g in range(BR // SUB):
                r0 = g * SUB
                hist = acc_ref[r0:r0 + SUB, :]

                def excl(colc):
                    e = colc == hist[:, 0:1]
                    for j in range(1, K_FULL):
                        e = e | (colc == hist[:, j:j + 1])
                    return e

                ms, cs = _build_cache(dist_ref, r0, lane_s, nc, excl_fn=excl)
                for j in range(CAP):
                    mtop_ref[j, r0:r0 + SUB, :] = ms[j]
                    ctop_ref[j, r0:r0 + SUB, :] = cs[j]
            mh_ref[...] = mtop_ref[0]
            ch_ref[...] = ctop_ref[0]
            ptr_ref[...] = jnp.zeros((BR, LANES), jnp.int32)

        mh = mh_ref[...]
        ch = ch_ref[...]
        m = jnp.min(mh, axis=1, keepdims=True)
        colpick = jnp.min(jnp.where(mh == m, ch, BIGC),
                          axis=1, keepdims=True)
        acc_ref[...] = jnp.where(iota32 == k, colpick, acc_ref[...])
        hit = ch == colpick
        ptrn = jnp.where(hit, ptr_ref[...] + 1, ptr_ref[...])
        sel_m = jnp.full((BR, LANES), jnp.inf, jnp.float32)
        sel_c = jnp.full((BR, LANES), BIGC, jnp.int32)
        for j in range(1, CAP):
            eqj = ptrn == j
            sel_m = jnp.where(eqj, mtop_ref[j], sel_m)
            sel_c = jnp.where(eqj, ctop_ref[j], sel_c)
        mh_ref[...] = jnp.where(hit, sel_m, mh)
        ch_ref[...] = jnp.where(hit, sel_c, ch)
        ptr_ref[...] = ptrn
        return carry

    jax.lax.fori_loop(0, K_FULL, ext_body, 0)

    acc = acc_ref[...]
    evens = jnp.concatenate([acc[:, 2 * j:2 * j + 1] for j in range(K_NB)],
                            axis=1)
    out_ref[...] = evens


@jax.jit
def kernel(x):
    n = x.shape[0]
    cstep = LANES * UNROLL
    np_cols = ((n + cstep - 1) // cstep) * cstep
    nr = ((n + BR - 1) // BR) * BR
    nc = np_cols // LANES
    xp = jnp.zeros((max(nr, np_cols), 3), x.dtype).at[:n].set(x)
    x_rows = xp[:nr]
    xt = xp[:np_cols].T

    grid = nr // BR
    nbr = pl.pallas_call(
        functools.partial(_knn_block, n_valid=n, np_cols=np_cols),
        grid=(grid,),
        in_specs=[
            pl.BlockSpec((BR, 3), lambda i: (i, 0)),
            pl.BlockSpec((3, np_cols), lambda i: (0, 0)),
        ],
        out_specs=pl.BlockSpec((BR, K_NB), lambda i: (i, 0)),
        out_shape=jax.ShapeDtypeStruct((nr, K_NB), jnp.int32),
        scratch_shapes=[
            pltpu.VMEM((nc, BR, LANES), jnp.float32),
            pltpu.VMEM((CAP, BR, LANES), jnp.float32),
            pltpu.VMEM((CAP, BR, LANES), jnp.int32),
            pltpu.VMEM((BR, LANES), jnp.float32),
            pltpu.VMEM((BR, LANES), jnp.int32),
            pltpu.VMEM((BR, LANES), jnp.int32),
            pltpu.VMEM((BR, K_FULL), jnp.int32),
        ],
        compiler_params=pltpu.CompilerParams(
            dimension_semantics=("parallel",),
        ),
    )(x_rows, xt)

    nbr = nbr[:n]
    center = jnp.broadcast_to(
        jnp.arange(n, dtype=jnp.int32)[:, None], (n, K_NB))
    return jnp.stack([nbr.reshape(-1), center.reshape(-1)], axis=0)


# speedup vs baseline: 11.9147x; 1.0570x over previous
import functools

import jax
import jax.numpy as jnp
from jax.experimental import pallas as pl
from jax.experimental.pallas import tpu as pltpu

K_NB = 16
K_FULL = 32
BR = 128
LANES = 128
CAP = 6
SUB = 8
UNROLL = 4
BIGC = 2**30


def _insert(d, colc, ms, cs):
    bs = [d < m for m in ms]
    nms, ncs = [], []
    for j in range(len(ms)):
        if j == 0:
            nms.append(jnp.where(bs[0], d, ms[0]))
            ncs.append(jnp.where(bs[0], colc, cs[0]))
        else:
            nms.append(jnp.where(bs[j], jnp.where(bs[j - 1], ms[j - 1], d), ms[j]))
            ncs.append(jnp.where(bs[j], jnp.where(bs[j - 1], cs[j - 1], colc), cs[j]))
    return tuple(nms), tuple(ncs)


def _build_cache2(dist_ref, r0a, r0b, lane_s, nc, excl_fns=None):
    def stream_body(i, carry):
        msa, csa, msb, csb = carry
        for u in range(UNROLL):
            c = i * UNROLL + u
            colc = c * LANES + lane_s
            da = dist_ref[c, r0a:r0a + SUB, :]
            db = dist_ref[c, r0b:r0b + SUB, :]
            if excl_fns is not None:
                da = jnp.where(excl_fns[0](colc), jnp.inf, da)
                db = jnp.where(excl_fns[1](colc), jnp.inf, db)
            msa, csa = _insert(da, colc, msa, csa)
            msb, csb = _insert(db, colc, msb, csb)
        return msa, csa, msb, csb

    ms0 = tuple(jnp.full((SUB, LANES), jnp.inf, jnp.float32)
                for _ in range(CAP))
    cs0 = tuple(jnp.full((SUB, LANES), BIGC, jnp.int32) for _ in range(CAP))
    return jax.lax.fori_loop(0, nc // UNROLL, stream_body,
                             (ms0, cs0, ms0, cs0))


def _knn_block(x_row_ref, xt_ref, out_ref, dist_ref, mtop_ref, ctop_ref,
               mh_ref, ch_ref, ptr_ref, acc_ref, *, n_valid, np_cols):
    nc = np_cols // LANES
    xr = x_row_ref[...]
    xt = xt_ref[...]

    sq_r = jnp.sum(xr * xr, axis=1, keepdims=True)
    sq_c = jnp.sum(xt * xt, axis=0, keepdims=True)
    s = jnp.dot(xr, xt, preferred_element_type=jnp.float32)

    col_full = jax.lax.broadcasted_iota(jnp.int32, (BR, np_cols), 1)
    d2 = (sq_r + sq_c) - 2.0 * s
    d2 = jnp.where(col_full < n_valid, d2, jnp.inf)
    for c in range(nc):
        dist_ref[c] = d2[:, c * LANES:(c + 1) * LANES]

    lane_s = jax.lax.broadcasted_iota(jnp.int32, (SUB, LANES), 1)
    for g in range(BR // (2 * SUB)):
        r0a, r0b = 2 * g * SUB, (2 * g + 1) * SUB
        msa, csa, msb, csb = _build_cache2(dist_ref, r0a, r0b, lane_s, nc)
        for j in range(CAP):
            mtop_ref[j, r0a:r0a + SUB, :] = msa[j]
            ctop_ref[j, r0a:r0a + SUB, :] = csa[j]
            mtop_ref[j, r0b:r0b + SUB, :] = msb[j]
            ctop_ref[j, r0b:r0b + SUB, :] = csb[j]

    mh_ref[...] = mtop_ref[0]
    ch_ref[...] = ctop_ref[0]
    ptr_ref[...] = jnp.zeros((BR, LANES), jnp.int32)
    acc_ref[...] = jnp.full((BR, K_FULL), -1, jnp.int32)

    iota32 = jax.lax.broadcasted_iota(jnp.int32, (BR, K_FULL), 1)

    def ext_body_checked(k, carry):
        exhausted = jnp.any(mh_ref[...] == jnp.inf)

        @pl.when(exhausted)
        def _():
            def make_excl(r0):
                hist = acc_ref[r0:r0 + SUB, :]

                def excl(colc):
                    e = colc == hist[:, 0:1]
                    for j in range(1, K_FULL):
                        e = e | (colc == hist[:, j:j + 1])
                    return e

                return excl

            for g in range(BR // (2 * SUB)):
                r0a, r0b = 2 * g * SUB, (2 * g + 1) * SUB
                msa, csa, msb, csb = _build_cache2(
                    dist_ref, r0a, r0b, lane_s, nc,
                    excl_fns=(make_excl(r0a), make_excl(r0b)))
                for j in range(CAP):
                    mtop_ref[j, r0a:r0a + SUB, :] = msa[j]
                    ctop_ref[j, r0a:r0a + SUB, :] = csa[j]
                    mtop_ref[j, r0b:r0b + SUB, :] = msb[j]
                    ctop_ref[j, r0b:r0b + SUB, :] = csb[j]
            mh_ref[...] = mtop_ref[0]
            ch_ref[...] = ctop_ref[0]
            ptr_ref[...] = jnp.zeros((BR, LANES), jnp.int32)

        _pop(k)
        return carry

    def ext_body_nocheck(k, carry):
        _pop(k)
        return carry

    def _pop(k):
        mh = mh_ref[...]
        ch = ch_ref[...]
        m = jnp.min(mh, axis=1, keepdims=True)
        colpick = jnp.min(jnp.where(mh == m, ch, BIGC),
                          axis=1, keepdims=True)
        acc_ref[...] = jnp.where(iota32 == k, colpick, acc_ref[...])
        hit = ch == colpick
        ptrn = jnp.where(hit, ptr_ref[...] + 1, ptr_ref[...])
        sel_m = jnp.full((BR, LANES), jnp.inf, jnp.float32)
        sel_c = jnp.full((BR, LANES), BIGC, jnp.int32)
        for j in range(1, CAP):
            eqj = ptrn == j
            sel_m = jnp.where(eqj, mtop_ref[j], sel_m)
            sel_c = jnp.where(eqj, ctop_ref[j], sel_c)
        mh_ref[...] = jnp.where(hit, sel_m, mh)
        ch_ref[...] = jnp.where(hit, sel_c, ch)
        ptr_ref[...] = ptrn

    jax.lax.fori_loop(0, CAP, ext_body_nocheck, 0)
    jax.lax.fori_loop(CAP, K_FULL, ext_body_checked, 0)

    acc = acc_ref[...]
    evens = jnp.concatenate([acc[:, 2 * j:2 * j + 1] for j in range(K_NB)],
                            axis=1)
    out_ref[...] = evens


@jax.jit
def kernel(x):
    n = x.shape[0]
    cstep = LANES * UNROLL
    np_cols = ((n + cstep - 1) // cstep) * cstep
    nr = ((n + BR - 1) // BR) * BR
    nc = np_cols // LANES
    xp = jnp.zeros((max(nr, np_cols), 3), x.dtype).at[:n].set(x)
    x_rows = xp[:nr]
    xt = xp[:np_cols].T

    grid = nr // BR
    nbr = pl.pallas_call(
        functools.partial(_knn_block, n_valid=n, np_cols=np_cols),
        grid=(grid,),
        in_specs=[
            pl.BlockSpec((BR, 3), lambda i: (i, 0)),
            pl.BlockSpec((3, np_cols), lambda i: (0, 0)),
        ],
        out_specs=pl.BlockSpec((BR, K_NB), lambda i: (i, 0)),
        out_shape=jax.ShapeDtypeStruct((nr, K_NB), jnp.int32),
        scratch_shapes=[
            pltpu.VMEM((nc, BR, LANES), jnp.float32),
            pltpu.VMEM((CAP, BR, LANES), jnp.float32),
            pltpu.VMEM((CAP, BR, LANES), jnp.int32),
            pltpu.VMEM((BR, LANES), jnp.float32),
            pltpu.VMEM((BR, LANES), jnp.int32),
            pltpu.VMEM((BR, LANES), jnp.int32),
            pltpu.VMEM((BR, K_FULL), jnp.int32),
        ],
        compiler_params=pltpu.CompilerParams(
            dimension_semantics=("parallel",),
        ),
    )(x_rows, xt)

    nbr = nbr[:n]
    center = jnp.broadcast_to(
        jnp.arange(n, dtype=jnp.int32)[:, None], (n, K_NB))
    return jnp.stack([nbr.reshape(-1), center.reshape(-1)], axis=0)
